# baseline (speedup 1.0000x reference)
import functools

import jax
import jax.numpy as jnp
from jax import lax
from jax.experimental import pallas as pl
from jax.experimental.pallas import tpu as pltpu
from jax.experimental.pallas import tpu_sc as plsc

_CH = 1024
_NBUF = 3


def _mxu_transpose(x, out_dim):
    eye = jnp.eye(out_dim, dtype=x.dtype)
    if x.shape[0] == out_dim:
        return lax.dot_general(x, eye, (((0,), (0,)), ((), ())),
                               precision=lax.Precision.HIGHEST)
    return lax.dot_general(eye, x, (((1,), (1,)), ((), ())),
                           precision=lax.Precision.HIGHEST)


@functools.cache
def _make_gather(Vt, D, N, B, F):
    info = plsc.get_sparse_core_info()
    NC, NS = info.num_cores, info.num_subcores
    NW = NC * NS
    assert N % (NW * _CH) == 0
    b_per_w = N // NW
    n_ch = b_per_w // _CH
    g = 128 // D
    S = B // g
    assert S % _CH == 0 and B % _CH == 0
    mesh = plsc.VectorSubcoreMesh(core_axis_name="c", subcore_axis_name="s")

    @functools.partial(
        pl.kernel,
        mesh=mesh,
        out_type=jax.ShapeDtypeStruct((N * D // 128, 128), jnp.float32),
        scratch_types=[
            pltpu.VMEM((n_ch, _CH), jnp.int32),
            pltpu.VMEM((_NBUF, _CH, D), jnp.float32),
        ]
        + [pltpu.SemaphoreType.DMA] * (2 * _NBUF),
        compiler_params=pltpu.CompilerParams(use_tc_tiling_on_sc=False),
    )
    def gather_kernel(idx_hbm, table_hbm, out_hbm, idx_all, rows, *sems):
        sem_g, sem_w = sems[:_NBUF], sems[_NBUF:]
        wid = lax.axis_index("s") * NC + lax.axis_index("c")
        base = wid * b_per_w
        pltpu.sync_copy(idx_hbm.at[wid], idx_all)

        def dest(i):
            pos = base + i * _CH
            f = pos // B
            t = pos % B
            j = t // S
            r = t % S
            return out_hbm.at[pl.ds(f * S + r, _CH), pl.ds(j * D, D)]

        gathers = {}
        for b in range(min(_NBUF, n_ch)):
            gathers[b] = pltpu.async_copy(
                table_hbm.at[idx_all.at[b]], rows.at[b], sem_g[b])
        for i in range(n_ch):
            b = i % _NBUF
            gathers[i].wait()
            wb = pltpu.async_copy(rows.at[b], dest(i), sem_w[b])
            nxt = i + _NBUF
            wb.wait()
            if nxt < n_ch:
                gathers[nxt] = pltpu.async_copy(
                    table_hbm.at[idx_all.at[nxt]], rows.at[b], sem_g[b])

    return gather_kernel


def _retile_body(in_ref, out_ref):
    x = in_ref[...]
    y = _mxu_transpose(x, x.shape[0])
    D = x.shape[0]
    g = 128 // D
    y3 = y.reshape(y.shape[0] // g, g, D)
    for j in range(g):
        out_ref[:, j * D:(j + 1) * D] = y3[:, j, :]


@functools.cache
def _make_retile(V, D):
    v_blk = 16384
    n_blk = -(-V // v_blk)
    R = v_blk * D // 128
    return pl.pallas_call(
        _retile_body,
        grid=(n_blk,),
        in_specs=[pl.BlockSpec((D, v_blk), lambda i: (0, i))],
        out_specs=pl.BlockSpec((R, 128), lambda i: (i, 0)),
        out_shape=jax.ShapeDtypeStruct((n_blk * R, 128), jnp.float32),
    )


def _unflatten_body(in_ref, out_ref):
    z = in_ref[...]
    D = out_ref.shape[1]
    g = 128 // D
    S = out_ref.shape[2] // g
    for j in range(g):
        out_ref[0, :, j * S:(j + 1) * S] = _mxu_transpose(
            z[:, j * D:(j + 1) * D], D)


@functools.cache
def _make_plane_transpose(B, F, D):
    rows = B * D // 128
    return pl.pallas_call(
        _unflatten_body,
        grid=(F,),
        in_specs=[pl.BlockSpec((rows, 128), lambda f: (f, 0))],
        out_specs=pl.BlockSpec((1, D, B), lambda f: (f, 0, 0)),
        out_shape=jax.ShapeDtypeStruct((F, D, B), jnp.float32),
    )


def kernel(token_ids, weight):
    B, F = token_ids.shape
    V, D = weight.shape
    N = B * F
    info = plsc.get_sparse_core_info()
    NW = info.num_cores * info.num_subcores
    idx = token_ids.T.reshape(NW, N // (NW * _CH), _CH)
    table = _make_retile(V, D)(weight.T)
    table = table.reshape(table.shape[0] * 128 // D, D)
    flat2 = _make_gather(table.shape[0], D, N, B, F)(idx, table)
    planes = _make_plane_transpose(B, F, D)(flat2)
    return planes.transpose(2, 0, 1)


# speedup vs baseline: 1.8295x; 1.8295x over previous
import functools

import jax
import jax.numpy as jnp
from jax import lax
from jax.experimental import pallas as pl
from jax.experimental.pallas import tpu as pltpu
from jax.experimental.pallas import tpu_sc as plsc

_CH = 1024
_NBUF = 3


def _mxu_transpose(x, out_dim):
    del out_dim
    return jnp.transpose(x, (1, 0))


@functools.cache
def _make_gather(Vt, D, N, B, F):
    info = plsc.get_sparse_core_info()
    NC, NS = info.num_cores, info.num_subcores
    NW = NC * NS
    assert N % (NW * _CH) == 0
    b_per_w = N // NW
    n_ch = b_per_w // _CH
    g = 128 // D
    S = B // g
    assert S % _CH == 0 and B % _CH == 0
    mesh = plsc.VectorSubcoreMesh(core_axis_name="c", subcore_axis_name="s")

    @functools.partial(
        pl.kernel,
        mesh=mesh,
        out_type=jax.ShapeDtypeStruct((N * D // 128, 128), jnp.float32),
        scratch_types=[
            pltpu.VMEM((n_ch, _CH), jnp.int32),
            pltpu.VMEM((_NBUF, _CH, D), jnp.float32),
        ]
        + [pltpu.SemaphoreType.DMA] * (2 * _NBUF),
        compiler_params=pltpu.CompilerParams(use_tc_tiling_on_sc=False),
    )
    def gather_kernel(idx_hbm, table_hbm, out_hbm, idx_all, rows, *sems):
        sem_g, sem_w = sems[:_NBUF], sems[_NBUF:]
        wid = lax.axis_index("s") * NC + lax.axis_index("c")
        base = wid * b_per_w
        pltpu.sync_copy(idx_hbm.at[wid], idx_all)

        def dest(i):
            pos = base + i * _CH
            f = pos // B
            t = pos % B
            j = t // S
            r = t % S
            return out_hbm.at[pl.ds(f * S + r, _CH), pl.ds(j * D, D)]

        gathers = {}
        for b in range(min(_NBUF, n_ch)):
            gathers[b] = pltpu.async_copy(
                table_hbm.at[idx_all.at[b]], rows.at[b], sem_g[b])
        for i in range(n_ch):
            b = i % _NBUF
            gathers[i].wait()
            wb = pltpu.async_copy(rows.at[b], dest(i), sem_w[b])
            nxt = i + _NBUF
            wb.wait()
            if nxt < n_ch:
                gathers[nxt] = pltpu.async_copy(
                    table_hbm.at[idx_all.at[nxt]], rows.at[b], sem_g[b])

    return gather_kernel


def _retile_body(in_ref, out_ref):
    x = in_ref[...]
    y = _mxu_transpose(x, x.shape[0])
    D = x.shape[0]
    g = 128 // D
    y3 = y.reshape(y.shape[0] // g, g, D)
    for j in range(g):
        out_ref[:, j * D:(j + 1) * D] = y3[:, j, :]


@functools.cache
def _make_retile(V, D):
    v_blk = 16384
    n_blk = -(-V // v_blk)
    R = v_blk * D // 128
    return pl.pallas_call(
        _retile_body,
        grid=(n_blk,),
        in_specs=[pl.BlockSpec((D, v_blk), lambda i: (0, i))],
        out_specs=pl.BlockSpec((R, 128), lambda i: (i, 0)),
        out_shape=jax.ShapeDtypeStruct((n_blk * R, 128), jnp.float32),
    )


def _unflatten_body(in_ref, out_ref):
    z = in_ref[...]
    D = out_ref.shape[1]
    g = 128 // D
    S = out_ref.shape[2] // g
    for j in range(g):
        out_ref[0, :, j * S:(j + 1) * S] = _mxu_transpose(
            z[:, j * D:(j + 1) * D], D)


@functools.cache
def _make_plane_transpose(B, F, D):
    rows = B * D // 128
    return pl.pallas_call(
        _unflatten_body,
        grid=(F,),
        in_specs=[pl.BlockSpec((rows, 128), lambda f: (f, 0))],
        out_specs=pl.BlockSpec((1, D, B), lambda f: (f, 0, 0)),
        out_shape=jax.ShapeDtypeStruct((F, D, B), jnp.float32),
    )


def kernel(token_ids, weight):
    B, F = token_ids.shape
    V, D = weight.shape
    N = B * F
    info = plsc.get_sparse_core_info()
    NW = info.num_cores * info.num_subcores
    idx = token_ids.T.reshape(NW, N // (NW * _CH), _CH)
    table = _make_retile(V, D)(weight.T)
    table = table.reshape(table.shape[0] * 128 // D, D)
    flat2 = _make_gather(table.shape[0], D, N, B, F)(idx, table)
    planes = _make_plane_transpose(B, F, D)(flat2)
    return planes.transpose(2, 0, 1)


# speedup vs baseline: 4.9467x; 2.7038x over previous
import functools

import jax
import jax.numpy as jnp
from jax import lax
from jax.experimental import pallas as pl
from jax.experimental.pallas import tpu as pltpu
from jax.experimental.pallas import tpu_sc as plsc

_CH = 1024
_NBUF = 3


@functools.cache
def _make_gather(Vt, D, N, B, F):
    info = plsc.get_sparse_core_info()
    NC, NS = info.num_cores, info.num_subcores
    NW = NC * NS
    assert N % (NW * _CH) == 0
    b_per_w = N // NW
    n_ch = b_per_w // _CH
    g = 128 // D
    S = B // g
    assert S % _CH == 0 and B % _CH == 0
    mesh = plsc.VectorSubcoreMesh(core_axis_name="c", subcore_axis_name="s")

    @functools.partial(
        pl.kernel,
        mesh=mesh,
        out_type=jax.ShapeDtypeStruct((N * D // 128, 128), jnp.float32),
        scratch_types=[
            pltpu.VMEM((n_ch, _CH), jnp.int32),
            pltpu.VMEM((_NBUF, _CH, D), jnp.float32),
        ]
        + [pltpu.SemaphoreType.DMA] * (2 * _NBUF),
        compiler_params=pltpu.CompilerParams(use_tc_tiling_on_sc=False),
    )
    def gather_kernel(idx_hbm, table_hbm, out_hbm, idx_all, rows, *sems):
        sem_g, sem_w = sems[:_NBUF], sems[_NBUF:]
        wid = lax.axis_index("s") * NC + lax.axis_index("c")
        base = wid * b_per_w
        pltpu.sync_copy(idx_hbm.at[wid], idx_all)

        def dest(i):
            pos = base + i * _CH
            f = pos // B
            t = pos % B
            j = t // S
            r = t % S
            return out_hbm.at[pl.ds(f * S + r, _CH), pl.ds(j * D, D)]

        gathers = {}
        for b in range(min(_NBUF, n_ch)):
            gathers[b] = pltpu.async_copy(
                table_hbm.at[idx_all.at[b]], rows.at[b], sem_g[b])
        for i in range(n_ch):
            b = i % _NBUF
            gathers[i].wait()
            wb = pltpu.async_copy(rows.at[b], dest(i), sem_w[b])
            nxt = i + _NBUF
            wb.wait()
            if nxt < n_ch:
                gathers[nxt] = pltpu.async_copy(
                    table_hbm.at[idx_all.at[nxt]], rows.at[b], sem_g[b])

    return gather_kernel


_W = 8192


def _retile_body(in_ref, out_ref):
    x = in_ref[...]
    D = x.shape[0]
    g = 128 // D
    xc = jnp.concatenate(
        [x[:, q * _W:(q + 1) * _W] for q in range(g)], axis=0)
    out_ref[...] = jnp.transpose(xc, (1, 0))


@functools.cache
def _make_retile(V, D):
    v_blk = (128 // D) * _W
    n_blk = -(-V // v_blk)
    return pl.pallas_call(
        _retile_body,
        grid=(n_blk,),
        in_specs=[pl.BlockSpec((D, v_blk), lambda i: (0, i))],
        out_specs=pl.BlockSpec((_W, 128), lambda i: (i, 0)),
        out_shape=jax.ShapeDtypeStruct((n_blk * _W, 128), jnp.float32),
    )


def _unflatten_body(in_ref, out_ref):
    z = in_ref[...]
    D = out_ref.shape[1]
    g = 128 // D
    S = out_ref.shape[2] // g
    zt = jnp.transpose(z, (1, 0))
    for j in range(g):
        out_ref[0, :, j * S:(j + 1) * S] = zt[j * D:(j + 1) * D, :]


@functools.cache
def _make_plane_transpose(B, F, D):
    rows = B * D // 128
    return pl.pallas_call(
        _unflatten_body,
        grid=(F,),
        in_specs=[pl.BlockSpec((rows, 128), lambda f: (f, 0))],
        out_specs=pl.BlockSpec((1, D, B), lambda f: (f, 0, 0)),
        out_shape=jax.ShapeDtypeStruct((F, D, B), jnp.float32),
    )


def kernel(token_ids, weight):
    B, F = token_ids.shape
    V, D = weight.shape
    N = B * F
    info = plsc.get_sparse_core_info()
    NW = info.num_cores * info.num_subcores
    tid = token_ids.T
    tid = ((tid >> 15) << 15) | ((tid & 8191) << 2) | ((tid >> 13) & 3)
    idx = tid.reshape(NW, N // (NW * _CH), _CH)
    table = _make_retile(V, D)(weight.T)
    table = table.reshape(table.shape[0] * 128 // D, D)
    flat2 = _make_gather(table.shape[0], D, N, B, F)(idx, table)
    planes = _make_plane_transpose(B, F, D)(flat2)
    return planes.transpose(2, 0, 1)


# speedup vs baseline: 5.0061x; 1.0120x over previous
import functools

import jax
import jax.numpy as jnp
from jax import lax
from jax.experimental import pallas as pl
from jax.experimental.pallas import tpu as pltpu
from jax.experimental.pallas import tpu_sc as plsc

_CH = 1024
_NBUF = 3


@functools.cache
def _make_gather(Vt, D, N, B, F):
    info = plsc.get_sparse_core_info()
    NC, NS = info.num_cores, info.num_subcores
    NW = NC * NS
    assert N % (NW * _CH) == 0
    b_per_w = N // NW
    n_ch = b_per_w // _CH
    g = 128 // D
    S = B // g
    assert S % _CH == 0 and B % _CH == 0
    mesh = plsc.VectorSubcoreMesh(core_axis_name="c", subcore_axis_name="s")

    @functools.partial(
        pl.kernel,
        mesh=mesh,
        out_type=jax.ShapeDtypeStruct((N * D // 128, 128), jnp.float32),
        scratch_types=[
            pltpu.VMEM((n_ch, _CH), jnp.int32),
            pltpu.VMEM((_NBUF, _CH, D), jnp.float32),
        ]
        + [pltpu.SemaphoreType.DMA] * (2 * _NBUF),
        compiler_params=pltpu.CompilerParams(use_tc_tiling_on_sc=False),
    )
    def gather_kernel(idx_hbm, table_hbm, out_hbm, idx_all, rows, *sems):
        sem_g, sem_w = sems[:_NBUF], sems[_NBUF:]
        wid = lax.axis_index("s") * NC + lax.axis_index("c")
        base = wid * b_per_w
        pltpu.sync_copy(idx_hbm.at[wid], idx_all)

        def dest(i):
            pos = base + i * _CH
            f = pos // B
            t = pos % B
            j = t // S
            r = t % S
            return out_hbm.at[pl.ds(f * S + r, _CH), pl.ds(j * D, D)]

        gathers = {}
        for b in range(min(_NBUF, n_ch)):
            gathers[b] = pltpu.async_copy(
                table_hbm.at[idx_all.at[b]], rows.at[b], sem_g[b])
        for i in range(n_ch):
            b = i % _NBUF
            gathers[i].wait()
            wb = pltpu.async_copy(rows.at[b], dest(i), sem_w[b])
            nxt = i + _NBUF
            wb.wait()
            if nxt < n_ch:
                gathers[nxt] = pltpu.async_copy(
                    table_hbm.at[idx_all.at[nxt]], rows.at[b], sem_g[b])

    return gather_kernel


_W = 8192


def _retile_body(in_ref, out_ref):
    x = in_ref[...]
    D = x.shape[0]
    g = 128 // D
    xc = jnp.concatenate(
        [x[:, q * _W:(q + 1) * _W] for q in range(g)], axis=0)
    out_ref[...] = jnp.transpose(xc, (1, 0))


@functools.cache
def _make_retile(V, D):
    v_blk = (128 // D) * _W
    n_blk = -(-V // v_blk)
    return pl.pallas_call(
        _retile_body,
        grid=(n_blk,),
        in_specs=[pl.BlockSpec((D, v_blk), lambda i: (0, i))],
        out_specs=pl.BlockSpec((_W, 128), lambda i: (i, 0)),
        out_shape=jax.ShapeDtypeStruct((n_blk * _W, 128), jnp.float32),
    )


def _unflatten_body(in_ref, out_ref):
    z = in_ref[...]
    D = out_ref.shape[1]
    g = 128 // D
    S = out_ref.shape[2] // g
    zt = jnp.transpose(z, (1, 0))
    for j in range(g):
        out_ref[0, :, j * S:(j + 1) * S] = zt[j * D:(j + 1) * D, :]


def _unflatten_alias_body(in_ref, prev_ref, out_ref):
    del prev_ref
    _unflatten_body(in_ref, out_ref)


@functools.cache
def _make_plane_transpose(B, F, D, f0, f_n):
    rows = B * D // 128
    if f0 == 0:
        return pl.pallas_call(
            _unflatten_body,
            grid=(f_n,),
            in_specs=[pl.BlockSpec((rows, 128), lambda f: (f, 0))],
            out_specs=pl.BlockSpec((1, D, B), lambda f: (f + f0, 0, 0)),
            out_shape=jax.ShapeDtypeStruct((F, D, B), jnp.float32),
        )
    return pl.pallas_call(
        _unflatten_alias_body,
        grid=(f_n,),
        in_specs=[
            pl.BlockSpec((rows, 128), lambda f: (f, 0)),
            pl.BlockSpec(memory_space=pltpu.MemorySpace.HBM),
        ],
        out_specs=pl.BlockSpec((1, D, B), lambda f: (f + f0, 0, 0)),
        out_shape=jax.ShapeDtypeStruct((F, D, B), jnp.float32),
        input_output_aliases={1: 0},
    )


def kernel(token_ids, weight):
    B, F = token_ids.shape
    V, D = weight.shape
    N = B * F
    info = plsc.get_sparse_core_info()
    NW = info.num_cores * info.num_subcores
    tid = token_ids.T
    tid = ((tid >> 15) << 15) | ((tid & 8191) << 2) | ((tid >> 13) & 3)
    table = _make_retile(V, D)(weight.T)
    table = table.reshape(table.shape[0] * 128 // D, D)
    FA = 16
    flats = []
    for f0, fn in ((0, FA), (FA, F - FA)):
        n_sub = fn * B
        idx = tid[f0:f0 + fn].reshape(NW, n_sub // (NW * _CH), _CH)
        flats.append(_make_gather(table.shape[0], D, n_sub, B, fn)(idx, table))
    planes = _make_plane_transpose(B, F, D, 0, FA)(flats[0])
    planes = _make_plane_transpose(B, F, D, FA, F - FA)(flats[1], planes)
    return planes.transpose(2, 0, 1)


# speedup vs baseline: 5.0436x; 1.0075x over previous
import functools

import jax
import jax.numpy as jnp
from jax import lax
from jax.experimental import pallas as pl
from jax.experimental.pallas import tpu as pltpu
from jax.experimental.pallas import tpu_sc as plsc

_CH = 1024
_NBUF = 3


@functools.cache
def _make_gather(Vt, D, N, B, F):
    info = plsc.get_sparse_core_info()
    NC, NS = info.num_cores, info.num_subcores
    NW = NC * NS
    assert N % (NW * _CH) == 0
    b_per_w = N // NW
    n_ch = b_per_w // _CH
    g = 128 // D
    S = B // g
    assert S % _CH == 0 and B % _CH == 0
    mesh = plsc.VectorSubcoreMesh(core_axis_name="c", subcore_axis_name="s")

    @functools.partial(
        pl.kernel,
        mesh=mesh,
        out_type=jax.ShapeDtypeStruct((N * D // 128, 128), jnp.float32),
        scratch_types=[
            pltpu.VMEM((n_ch, _CH), jnp.int32),
            pltpu.VMEM((_NBUF, _CH, D), jnp.float32),
        ]
        + [pltpu.SemaphoreType.DMA] * (2 * _NBUF),
        compiler_params=pltpu.CompilerParams(use_tc_tiling_on_sc=False),
    )
    def gather_kernel(idx_hbm, table_hbm, out_hbm, idx_all, rows, *sems):
        sem_g, sem_w = sems[:_NBUF], sems[_NBUF:]
        wid = lax.axis_index("s") * NC + lax.axis_index("c")
        base = wid * b_per_w
        pltpu.sync_copy(idx_hbm.at[wid], idx_all)

        def dest(i):
            pos = base + i * _CH
            f = pos // B
            t = pos % B
            j = t // S
            r = t % S
            return out_hbm.at[pl.ds(f * S + r, _CH), pl.ds(j * D, D)]

        gathers = {}
        for b in range(min(_NBUF, n_ch)):
            gathers[b] = pltpu.async_copy(
                table_hbm.at[idx_all.at[b]], rows.at[b], sem_g[b])
        for i in range(n_ch):
            b = i % _NBUF
            gathers[i].wait()
            wb = pltpu.async_copy(rows.at[b], dest(i), sem_w[b])
            nxt = i + _NBUF
            wb.wait()
            if nxt < n_ch:
                gathers[nxt] = pltpu.async_copy(
                    table_hbm.at[idx_all.at[nxt]], rows.at[b], sem_g[b])

    return gather_kernel


_W = 16384


def _retile_body(in_ref, out_ref):
    x = in_ref[...]
    D = x.shape[0]
    g = 128 // D
    xc = jnp.concatenate(
        [x[:, q * _W:(q + 1) * _W] for q in range(g)], axis=0)
    out_ref[...] = jnp.transpose(xc, (1, 0))


@functools.cache
def _make_retile(V, D):
    v_blk = (128 // D) * _W
    n_blk = -(-V // v_blk)
    return pl.pallas_call(
        _retile_body,
        grid=(n_blk,),
        in_specs=[pl.BlockSpec((D, v_blk), lambda i: (0, i))],
        out_specs=pl.BlockSpec((_W, 128), lambda i: (i, 0)),
        out_shape=jax.ShapeDtypeStruct((n_blk * _W, 128), jnp.float32),
    )


def _unflatten_body(in_ref, out_ref):
    z = in_ref[...]
    D = out_ref.shape[1]
    g = 128 // D
    S = out_ref.shape[2] // g
    zt = jnp.transpose(z, (1, 0))
    for j in range(g):
        out_ref[0, :, j * S:(j + 1) * S] = zt[j * D:(j + 1) * D, :]


def _unflatten_alias_body(in_ref, prev_ref, out_ref):
    del prev_ref
    _unflatten_body(in_ref, out_ref)


@functools.cache
def _make_plane_transpose(B, F, D, f0, f_n):
    rows = B * D // 128
    if f0 == 0:
        return pl.pallas_call(
            _unflatten_body,
            grid=(f_n,),
            in_specs=[pl.BlockSpec((rows, 128), lambda f: (f, 0))],
            out_specs=pl.BlockSpec((1, D, B), lambda f: (f + f0, 0, 0)),
            out_shape=jax.ShapeDtypeStruct((F, D, B), jnp.float32),
        )
    return pl.pallas_call(
        _unflatten_alias_body,
        grid=(f_n,),
        in_specs=[
            pl.BlockSpec((rows, 128), lambda f: (f, 0)),
            pl.BlockSpec(memory_space=pltpu.MemorySpace.HBM),
        ],
        out_specs=pl.BlockSpec((1, D, B), lambda f: (f + f0, 0, 0)),
        out_shape=jax.ShapeDtypeStruct((F, D, B), jnp.float32),
        input_output_aliases={1: 0},
    )


def kernel(token_ids, weight):
    B, F = token_ids.shape
    V, D = weight.shape
    N = B * F
    info = plsc.get_sparse_core_info()
    NW = info.num_cores * info.num_subcores
    g = 128 // D
    lw = _W.bit_length() - 1
    lgw = (g * _W).bit_length() - 1
    tid = token_ids.T
    tid = ((tid >> lgw) << lgw) | ((tid & (_W - 1)) << 2) | ((tid >> lw) & (g - 1))
    table = _make_retile(V, D)(weight.T)
    table = table.reshape(table.shape[0] * 128 // D, D)
    FA = 16
    flats = []
    for f0, fn in ((0, FA), (FA, F - FA)):
        n_sub = fn * B
        idx = tid[f0:f0 + fn].reshape(NW, n_sub // (NW * _CH), _CH)
        flats.append(_make_gather(table.shape[0], D, n_sub, B, fn)(idx, table))
    planes = _make_plane_transpose(B, F, D, 0, FA)(flats[0])
    planes = _make_plane_transpose(B, F, D, FA, F - FA)(flats[1], planes)
    return planes.transpose(2, 0, 1)
